# Initial kernel scaffold; baseline (speedup 1.0000x reference)
import jax
import jax.numpy as jnp
from jax.experimental import pallas as pl


def kernel(x_0, x_1, x_2, neighborhood_1_to_1, neighborhood_2_to_1, neighborhood_0_to_1, W_1to1, W_2to1, W_0to1, W_update, b_update):
    raise NotImplementedError("write your pallas kernel here")


# speedup vs baseline: 1.0788x; 1.0788x over previous
import functools

import jax
import jax.numpy as jnp
from jax.experimental import pallas as pl
from jax.experimental.pallas import tpu as pltpu


def _elu(x):
    return jnp.where(x > 0, x, jnp.exp(x) - 1.0)


def _proj_kernel(x0_ref, x1_ref, x2_ref, w0_ref, w1_ref, w2_ref,
                 o0_ref, o1_ref, o2_ref):
    o0_ref[...] = jnp.dot(x0_ref[...], w0_ref[...],
                          preferred_element_type=jnp.float32)
    o1_ref[...] = jnp.dot(x1_ref[...], w1_ref[...],
                          preferred_element_type=jnp.float32)
    o2_ref[...] = jnp.dot(x2_ref[...], w2_ref[...],
                          preferred_element_type=jnp.float32)


def _main_kernel(n11_ref, n21_ref, n01_ref, xw1_ref, xw2_ref, xw0_ref,
                 wu_ref, bu_ref, out_ref):
    a = _elu(jnp.dot(n11_ref[...], xw1_ref[...],
                     preferred_element_type=jnp.float32))
    b = _elu(jnp.dot(n21_ref[...], xw2_ref[...],
                     preferred_element_type=jnp.float32))
    c = _elu(jnp.dot(n01_ref[...], xw0_ref[...],
                     preferred_element_type=jnp.float32))
    agg = a + b + c
    out_ref[...] = _elu(jnp.dot(agg, wu_ref[...],
                                preferred_element_type=jnp.float32)
                        + bu_ref[...])


@functools.partial(jax.jit, static_argnames=("block_m",))
def _cwn_forward(x_0, x_1, x_2, n11, n21, n01, w1, w2, w0, wu, bu,
                 block_m=256):
    n_rows, d = x_1.shape
    dout = wu.shape[1]

    xw0, xw1, xw2 = pl.pallas_call(
        _proj_kernel,
        out_shape=[jax.ShapeDtypeStruct((x_0.shape[0], dout), jnp.float32),
                   jax.ShapeDtypeStruct((x_1.shape[0], dout), jnp.float32),
                   jax.ShapeDtypeStruct((x_2.shape[0], dout), jnp.float32)],
    )(x_0, x_1, x_2, w0, w1, w2)

    k1 = n11.shape[1]
    k2 = n21.shape[1]
    k0 = n01.shape[1]
    grid = (n_rows // block_m,)
    out = pl.pallas_call(
        _main_kernel,
        grid=grid,
        in_specs=[
            pl.BlockSpec((block_m, k1), lambda i: (i, 0)),
            pl.BlockSpec((block_m, k2), lambda i: (i, 0)),
            pl.BlockSpec((block_m, k0), lambda i: (i, 0)),
            pl.BlockSpec((k1, dout), lambda i: (0, 0)),
            pl.BlockSpec((k2, dout), lambda i: (0, 0)),
            pl.BlockSpec((k0, dout), lambda i: (0, 0)),
            pl.BlockSpec((dout, dout), lambda i: (0, 0)),
            pl.BlockSpec((1, dout), lambda i: (0, 0)),
        ],
        out_specs=pl.BlockSpec((block_m, dout), lambda i: (i, 0)),
        out_shape=jax.ShapeDtypeStruct((n_rows, dout), jnp.float32),
        compiler_params=pltpu.CompilerParams(
            vmem_limit_bytes=100 * 1024 * 1024),
    )(n11, n21, n01, xw1, xw2, xw0, wu, bu.reshape(1, dout))
    return out


def kernel(x_0, x_1, x_2, neighborhood_1_to_1, neighborhood_2_to_1,
           neighborhood_0_to_1, W_1to1, W_2to1, W_0to1, W_update, b_update):
    return _cwn_forward(x_0, x_1, x_2, neighborhood_1_to_1,
                        neighborhood_2_to_1, neighborhood_0_to_1,
                        W_1to1, W_2to1, W_0to1, W_update, b_update)


# speedup vs baseline: 1.1248x; 1.0426x over previous
import functools

import jax
import jax.numpy as jnp
from jax.experimental import pallas as pl
from jax.experimental.pallas import tpu as pltpu


def _elu(x):
    return jnp.where(x > 0, x, jnp.exp(x) - 1.0)


def _fused_kernel(n11_ref, n21_ref, n01_ref, x1_ref, x2_ref, x0_ref,
                  w1_ref, w2_ref, w0_ref, wu_ref, bu_ref,
                  out_ref, xw1_s, xw2_s, xw0_s):
    @pl.when(pl.program_id(0) == 0)
    def _project():
        xw1_s[...] = jnp.dot(x1_ref[...], w1_ref[...],
                             preferred_element_type=jnp.float32)
        xw2_s[...] = jnp.dot(x2_ref[...], w2_ref[...],
                             preferred_element_type=jnp.float32)
        xw0_s[...] = jnp.dot(x0_ref[...], w0_ref[...],
                             preferred_element_type=jnp.float32)

    a = _elu(jnp.dot(n11_ref[...], xw1_s[...],
                     preferred_element_type=jnp.float32))
    b = _elu(jnp.dot(n21_ref[...], xw2_s[...],
                     preferred_element_type=jnp.float32))
    c = _elu(jnp.dot(n01_ref[...], xw0_s[...],
                     preferred_element_type=jnp.float32))
    agg = a + b + c
    out_ref[...] = _elu(jnp.dot(agg, wu_ref[...],
                                preferred_element_type=jnp.float32)
                        + bu_ref[...])


@functools.partial(jax.jit, static_argnames=("block_m",))
def _cwn_forward(x_0, x_1, x_2, n11, n21, n01, w1, w2, w0, wu, bu,
                 block_m=128):
    n_rows, d = x_1.shape
    dout = wu.shape[1]
    k1 = n11.shape[1]
    k2 = n21.shape[1]
    k0 = n01.shape[1]
    grid = (n_rows // block_m,)
    out = pl.pallas_call(
        _fused_kernel,
        grid=grid,
        in_specs=[
            pl.BlockSpec((block_m, k1), lambda i: (i, 0)),
            pl.BlockSpec((block_m, k2), lambda i: (i, 0)),
            pl.BlockSpec((block_m, k0), lambda i: (i, 0)),
            pl.BlockSpec((x_1.shape[0], d), lambda i: (0, 0)),
            pl.BlockSpec((x_2.shape[0], d), lambda i: (0, 0)),
            pl.BlockSpec((x_0.shape[0], d), lambda i: (0, 0)),
            pl.BlockSpec((d, dout), lambda i: (0, 0)),
            pl.BlockSpec((d, dout), lambda i: (0, 0)),
            pl.BlockSpec((d, dout), lambda i: (0, 0)),
            pl.BlockSpec((dout, dout), lambda i: (0, 0)),
            pl.BlockSpec((1, dout), lambda i: (0, 0)),
        ],
        out_specs=pl.BlockSpec((block_m, dout), lambda i: (i, 0)),
        out_shape=jax.ShapeDtypeStruct((n_rows, dout), jnp.float32),
        scratch_shapes=[
            pltpu.VMEM((k1, dout), jnp.float32),
            pltpu.VMEM((k2, dout), jnp.float32),
            pltpu.VMEM((k0, dout), jnp.float32),
        ],
        compiler_params=pltpu.CompilerParams(
            vmem_limit_bytes=63 * 1024 * 1024),
    )(n11, n21, n01, x_1, x_2, x_0, w1, w2, w0, wu, bu.reshape(1, dout))
    return out


def kernel(x_0, x_1, x_2, neighborhood_1_to_1, neighborhood_2_to_1,
           neighborhood_0_to_1, W_1to1, W_2to1, W_0to1, W_update, b_update):
    return _cwn_forward(x_0, x_1, x_2, neighborhood_1_to_1,
                        neighborhood_2_to_1, neighborhood_0_to_1,
                        W_1to1, W_2to1, W_0to1, W_update, b_update)


# speedup vs baseline: 1.1443x; 1.0174x over previous
import functools

import jax
import jax.numpy as jnp
from jax.experimental import pallas as pl
from jax.experimental.pallas import tpu as pltpu


def _elu(x):
    return jnp.where(x > 0, x, jnp.exp(x) - 1.0)


def _fused_kernel(n11_ref, n21_ref, n01_ref, x1_ref, x2_ref, x0_ref,
                  w1_ref, w2_ref, w0_ref, wu_ref, bu_ref,
                  out_ref, xw1_s, xw2_s, xw0_s):
    @pl.when(pl.program_id(0) == 0)
    def _project():
        xw1_s[...] = jnp.dot(x1_ref[...], w1_ref[...],
                             preferred_element_type=jnp.float32)
        xw2_s[...] = jnp.dot(x2_ref[...], w2_ref[...],
                             preferred_element_type=jnp.float32)
        xw0_s[...] = jnp.dot(x0_ref[...], w0_ref[...],
                             preferred_element_type=jnp.float32)

    a = _elu(jnp.dot(n11_ref[...], xw1_s[...],
                     preferred_element_type=jnp.float32))
    b = _elu(jnp.dot(n21_ref[...], xw2_s[...],
                     preferred_element_type=jnp.float32))
    c = _elu(jnp.dot(n01_ref[...], xw0_s[...],
                     preferred_element_type=jnp.float32))
    agg = a + b + c
    out_ref[...] = _elu(jnp.dot(agg, wu_ref[...],
                                preferred_element_type=jnp.float32)
                        + bu_ref[...])


@functools.partial(jax.jit, static_argnames=("block_m",))
def _cwn_forward(x_0, x_1, x_2, n11, n21, n01, w1, w2, w0, wu, bu,
                 block_m=192):
    n_rows, d = x_1.shape
    dout = wu.shape[1]
    k1 = n11.shape[1]
    k2 = n21.shape[1]
    k0 = n01.shape[1]
    grid = (n_rows // block_m,)
    out = pl.pallas_call(
        _fused_kernel,
        grid=grid,
        in_specs=[
            pl.BlockSpec((block_m, k1), lambda i: (i, 0)),
            pl.BlockSpec((block_m, k2), lambda i: (i, 0)),
            pl.BlockSpec((block_m, k0), lambda i: (i, 0)),
            pl.BlockSpec((x_1.shape[0], d), lambda i: (0, 0)),
            pl.BlockSpec((x_2.shape[0], d), lambda i: (0, 0)),
            pl.BlockSpec((x_0.shape[0], d), lambda i: (0, 0)),
            pl.BlockSpec((d, dout), lambda i: (0, 0)),
            pl.BlockSpec((d, dout), lambda i: (0, 0)),
            pl.BlockSpec((d, dout), lambda i: (0, 0)),
            pl.BlockSpec((dout, dout), lambda i: (0, 0)),
            pl.BlockSpec((1, dout), lambda i: (0, 0)),
        ],
        out_specs=pl.BlockSpec((block_m, dout), lambda i: (i, 0)),
        out_shape=jax.ShapeDtypeStruct((n_rows, dout), jnp.float32),
        scratch_shapes=[
            pltpu.VMEM((k1, dout), jnp.float32),
            pltpu.VMEM((k2, dout), jnp.float32),
            pltpu.VMEM((k0, dout), jnp.float32),
        ],
        compiler_params=pltpu.CompilerParams(
            vmem_limit_bytes=63 * 1024 * 1024),
    )(n11, n21, n01, x_1, x_2, x_0, w1, w2, w0, wu, bu.reshape(1, dout))
    return out


def kernel(x_0, x_1, x_2, neighborhood_1_to_1, neighborhood_2_to_1,
           neighborhood_0_to_1, W_1to1, W_2to1, W_0to1, W_update, b_update):
    return _cwn_forward(x_0, x_1, x_2, neighborhood_1_to_1,
                        neighborhood_2_to_1, neighborhood_0_to_1,
                        W_1to1, W_2to1, W_0to1, W_update, b_update)
